# baseline (speedup 1.0000x reference)
import jax
import jax.numpy as jnp
from jax import lax
from jax.experimental import pallas as pl
from jax.experimental.pallas import tpu as pltpu
from jax.experimental.pallas import tpu_sc as plsc

_SCALES = (1.0, 2.0, 4.0)
_PADS = (3, 6, 12)
_INV2S2 = tuple(1.0 / (2.0 * s * s) for s in _SCALES)
_NUM_KP = 15
_B = 8
_H = 384
_W = 384
_ZROWS = 24
_NC = 2
_NS = 16
_PLANES = _B * _NUM_KP
_PPC = _PLANES // _NC
_PROWS = 32


def _sc_body(kp_hbm, out_hbm, kp_v, zero_v, patch_v, sem):
    core = lax.axis_index("c")
    sub = lax.axis_index("s")
    lane = lax.broadcasted_iota(jnp.int32, (16,), 0)
    zvec = jnp.zeros((16,), jnp.float32)

    pltpu.sync_copy(kp_hbm, kp_v)

    def zrow(r, c):
        for h in range(_W // 16):
            zero_v[r, pl.ds(h * 16, 16)] = zvec
        return c

    lax.fori_loop(0, _ZROWS, zrow, 0)

    wvi = kp_v[pl.ds(240, 16)]
    wpos = [jnp.maximum(lax.bitcast_convert_type(wvi[s], jnp.float32), 0.0)
            for s in range(3)]

    def _zero_cols(xa):
        def body(r, c):
            for h in range(3):
                patch_v[r, pl.ds(xa + 16 * h, 16)] = zvec
            return c

        lax.fori_loop(0, _PROWS, body, 0)

    for j in range(4):
        idx = sub + _NS * j

        @pl.when(idx < _PPC)
        def _():
            p = core * _PPC + idx
            b, k = p // _NUM_KP, p % _NUM_KP

            def fire(seg, c):
                dst = out_hbm.at[b, k, pl.ds(seg * _ZROWS, _ZROWS), :]
                pltpu.async_copy(zero_v, dst, sem)
                return c

            lax.fori_loop(0, _H // _ZROWS, fire, 0)

            if j == 0:
                def pzero(r, c):
                    for h in range(_W // 16):
                        patch_v[r, pl.ds(h * 16, 16)] = zvec
                    return c

                lax.fori_loop(0, _PROWS, pzero, 0)

            va = kp_v[pl.ds(2 * p, 16)]
            x = va[0]
            y = va[1]
            xs = jnp.clip(x - 12, 0, _W - 25)
            ys = jnp.clip(y - 12, 0, _H - 25)
            xa = (xs // 16) * 16
            ya = (ys // 8) * 8

            a = []
            for s in range(3):
                pad = _PADS[s]
                ok = ((x >= pad) & (x < _W - pad)
                      & (y >= pad) & (y < _H - pad))
                a.append(wpos[s] * ok.astype(jnp.float32))

            fx = []
            cy = []
            for s in range(3):
                pad = _PADS[s]
                inv = _INV2S2[s]
                fxs = []
                for h in range(3):
                    dxv = xa + 16 * h + lane - x
                    dx2 = (dxv * dxv).astype(jnp.float32)
                    fxs.append(jnp.where(jnp.abs(dxv) <= pad,
                                         jnp.exp(-dx2 * inv), 0.0))
                fx.append(fxs)
                cys = []
                for h in range(2):
                    dyv = ya + 16 * h + lane - y
                    dy2 = (dyv * dyv).astype(jnp.float32)
                    cys.append(jnp.where(jnp.abs(dyv) <= pad,
                                         jnp.exp(-dy2 * inv), 0.0) * a[s])
                cy.append(cys)

            def row_body(r, carry):
                ln = jnp.full((16,), r % 16, jnp.int32)
                lo = r < 16
                cv = [jnp.where(lo, cy[s][0], cy[s][1])
                      .at[ln].get(mode="promise_in_bounds")
                      for s in range(3)]
                for h in range(3):
                    v = jnp.maximum(
                        jnp.maximum(cv[0] * fx[0][h], cv[1] * fx[1][h]),
                        cv[2] * fx[2][h])
                    patch_v[r, pl.ds(xa + 16 * h, 16)] = v
                return carry

            lax.fori_loop(0, _PROWS, row_body, 0)

            def drain(seg, c):
                dst = out_hbm.at[b, k, pl.ds(seg * _ZROWS, _ZROWS), :]
                pltpu.make_async_copy(zero_v, dst, sem).wait()
                return c

            lax.fori_loop(0, _H // _ZROWS, drain, 0)

            pltpu.sync_copy(patch_v, out_hbm.at[b, k, pl.ds(ya, _PROWS), :])
            if j < 3:
                _zero_cols(xa)


def kernel(image_tensor, keypoints, scale_weights):
    B, _, H, W = image_tensor.shape
    packed = jnp.concatenate([
        keypoints.astype(jnp.int32).reshape(-1),
        lax.bitcast_convert_type(scale_weights.astype(jnp.float32),
                                 jnp.int32),
        jnp.zeros((13,), jnp.int32),
    ])
    mesh = plsc.VectorSubcoreMesh(
        core_axis_name="c", subcore_axis_name="s",
        num_cores=_NC, num_subcores=_NS)
    f = pl.kernel(
        _sc_body,
        out_type=jax.ShapeDtypeStruct((B, _NUM_KP, H, W), jnp.float32),
        mesh=mesh,
        scratch_types=[
            pltpu.VMEM((256,), jnp.int32),
            pltpu.VMEM((_ZROWS, _W), jnp.float32),
            pltpu.VMEM((_PROWS, _W), jnp.float32),
            pltpu.SemaphoreType.DMA,
        ],
        compiler_params=pltpu.CompilerParams(use_tc_tiling_on_sc=True),
    )
    return f(packed)
